# Initial kernel scaffold; baseline (speedup 1.0000x reference)
import jax
import jax.numpy as jnp
from jax.experimental import pallas as pl


def kernel(x, W1, b1, W2, b2, pw1, pb1, pw2, pb2, hw1, hb1, hw2, hb2, S_m):
    raise NotImplementedError("write your pallas kernel here")


# speedup vs baseline: 26.6279x; 26.6279x over previous
import functools

import jax
import jax.numpy as jnp
from jax import lax
from jax.experimental import pallas as pl
from jax.experimental.pallas import tpu as pltpu
from jax.experimental.pallas import tpu_sc as plsc

N = 4096
C = 128
K = 32
G = 64
WIN = 5
NW = 2 * WIN + 1
NWP = 16
NCAND = NW * NWP
NKNOTS = 32

NBLK_A = 128
EBLK = 2048
NBLK_C = EBLK // K

SC_NC = 2
SC_NS = 16
SC_NWORK = SC_NC * SC_NS
SC_CH = 128


def _sel_kernel(dy2_ref, dx2_ref, idx_ref, r_ref):
    pid = pl.program_id(0)
    row = lax.broadcasted_iota(jnp.int32, (NBLK_A, NWP), 0) + pid * NBLK_A
    lane = lax.broadcasted_iota(jnp.int32, (NBLK_A, NWP), 1)
    dx2 = dx2_ref[...]
    dpieces = []
    jpieces = []
    for dyi in range(NW):
        dpieces.append(jnp.sqrt(dy2_ref[:, dyi:dyi + 1] + dx2))
        jpieces.append(row + ((dyi - WIN) * G - WIN) + lane)
    dist = jnp.concatenate(dpieces, axis=1)
    jmat = jnp.concatenate(jpieces, axis=1)
    big = jnp.int32(2 ** 30)
    for s in range(K):
        m = jnp.min(dist, axis=1, keepdims=True)
        jm = jnp.min(jnp.where(dist == m, jmat, big), axis=1, keepdims=True)
        idx_ref[:, s:s + 1] = jm
        r_ref[:, s:s + 1] = m
        dist = jnp.where(jmat == jm, jnp.inf, dist)


def _edge_kernel(idx_ref, r_ref, pw1_ref, pb1_ref, pw2_ref, pb2_ref,
                 hw1_ref, hb1_ref, hw2_ref, hb2_ref, sm_ref, sms_ref,
                 phi_ref, psi_ref, pacc_ref, phacc_ref):
    pid = pl.program_id(0)
    step = jnp.float32(1.0 / (G - 1))
    e = lax.broadcasted_iota(jnp.int32, (EBLK, 1), 0) + pid * EBLK
    n = e // K
    iy = (n // G).astype(jnp.float32) * step
    ix = (n % G).astype(jnp.float32) * step
    j = idx_ref[...]
    jy = (j // G).astype(jnp.float32) * step
    jx = (j % G).astype(jnp.float32) * step

    hid = jax.nn.relu(iy * pw1_ref[0:1, :] + ix * pw1_ref[1:2, :]
                      + jy * pw1_ref[2:3, :] + jx * pw1_ref[3:4, :]
                      + pb1_ref[...])
    phi = lax.dot_general(hid, pw2_ref[...], (((1,), (0,)), ((), ())),
                          preferred_element_type=jnp.float32) + pb2_ref[...]
    phi_ref[...] = phi

    hidh = jax.nn.relu(iy * hw1_ref[0:1, :] + ix * hw1_ref[1:2, :]
                       + hb1_ref[...])
    hsum = jnp.sum(hidh * hw2_ref[...], axis=1, keepdims=True) + hb2_ref[0, 0]
    h = jnp.maximum(hsum, 0.0) + jnp.log1p(jnp.exp(-jnp.abs(hsum)))

    rsc = jnp.clip(r_ref[...] / (h + 1e-6), 0.0, 1.0)
    kstep = jnp.float32(1.0 / (NKNOTS - 1))
    binf = jnp.clip(jnp.floor(rsc * (NKNOTS - 1)), 0.0, jnp.float32(NKNOTS - 2))
    bini = binf.astype(jnp.int32)
    lane32 = lax.broadcasted_iota(jnp.int32, (EBLK, NKNOTS), 1)
    onehot = (bini == lane32).astype(jnp.float32)
    w_k = jnp.sum(onehot * sm_ref[...], axis=1, keepdims=True)
    w_k1 = jnp.sum(onehot * sms_ref[...], axis=1, keepdims=True)
    t_k = binf * kstep
    t_k1 = (binf + 1.0) * kstep
    wr = (rsc - t_k) / (t_k1 - t_k + 1e-8)
    psi = (1.0 - wr) * w_k + wr * w_k1
    psi_ref[...] = psi

    @pl.when(pid == 0)
    def _():
        pacc_ref[...] = jnp.zeros_like(pacc_ref)
        phacc_ref[...] = jnp.zeros_like(phacc_ref)

    pacc_ref[...] += jnp.sum(jnp.abs(psi))
    phacc_ref[...] += jnp.sum(jnp.abs(phi), axis=0, keepdims=True)


def _agg_kernel(u0_ref, u1_ref, phi_ref, psi_ref, pacc_ref, phacc_ref,
                x_ref, w1_ref, b1_ref, w2_ref, b2_ref, out_ref):
    nedge = jnp.float32(N * K)
    psi_scale = 1.0 / (pacc_ref[0, 0] / nedge + 1e-6)
    phi_scale = 1.0 / (phacc_ref[...] / nedge + 1e-6)
    w_edge = (psi_ref[...] * psi_scale) * (phi_ref[...] * phi_scale)
    a0 = jnp.sum((w_edge * u0_ref[...]).reshape(NBLK_C, K, C), axis=1)
    a1 = jnp.sum((w_edge * u1_ref[...]).reshape(NBLK_C, K, C), axis=1)
    inv_k = jnp.float32(1.0 / K)

    xb = x_ref[...].reshape(2 * NBLK_C, C)
    hid = jax.nn.relu(
        lax.dot_general(xb, w1_ref[...], (((1,), (0,)), ((), ())),
                        preferred_element_type=jnp.float32) + b1_ref[...])
    pw = lax.dot_general(hid, w2_ref[...], (((1,), (0,)), ((), ())),
                         preferred_element_type=jnp.float32) + b2_ref[...]
    out_ref[0] = a0 * inv_k + pw[:NBLK_C]
    out_ref[1] = a1 * inv_k + pw[NBLK_C:]


def _sc_gather(xflat, idxcat):
    nrows = idxcat.shape[0]
    per_w = nrows // SC_NWORK
    nchunk = per_w // SC_CH
    mesh = plsc.VectorSubcoreMesh(core_axis_name="c", subcore_axis_name="s")

    @functools.partial(
        pl.kernel, mesh=mesh,
        out_type=jax.ShapeDtypeStruct((nrows, C), jnp.float32),
        scratch_types=[
            pltpu.VMEM((SC_CH,), jnp.int32),
            pltpu.VMEM((SC_CH, C), jnp.float32),
            pltpu.SemaphoreType.DMA,
        ],
    )
    def gather_k(x_hbm, idx_hbm, u_hbm, idx_v, rows_v, sem):
        wid = lax.axis_index("s") * SC_NC + lax.axis_index("c")
        base = wid * per_w

        def body(ci, carry):
            cb = base + ci * SC_CH
            pltpu.sync_copy(idx_hbm.at[pl.ds(cb, SC_CH)], idx_v)
            pltpu.async_copy(x_hbm.at[idx_v], rows_v, sem).wait()
            pltpu.sync_copy(rows_v, u_hbm.at[pl.ds(cb, SC_CH)])
            return carry

        lax.fori_loop(0, nchunk, body, 0)

    return gather_k(xflat, idxcat)


def _neighbor_tables():
    cy = jnp.linspace(0.0, 1.0, G).astype(jnp.float32)
    offs = jnp.arange(NW) - WIN
    a = jnp.arange(G)
    nb = a[:, None] + offs[None, :]
    valid = (nb >= 0) & (nb < G)
    d = cy[:, None] - cy[jnp.clip(nb, 0, G - 1)]
    d2 = jnp.where(valid, d * d, jnp.inf)
    d2 = jnp.concatenate(
        [d2, jnp.full((G, NWP - NW), jnp.inf, jnp.float32)], axis=1)
    nodes = jnp.arange(N)
    return d2[nodes // G], d2[nodes % G]


def kernel(x, W1, b1, W2, b2, pw1, pb1, pw2, pb2, hw1, hb1, hw2, hb2, S_m):
    dy2, dx2 = _neighbor_tables()

    idx, r = pl.pallas_call(
        _sel_kernel,
        grid=(N // NBLK_A,),
        in_specs=[
            pl.BlockSpec((NBLK_A, NWP), lambda i: (i, 0)),
            pl.BlockSpec((NBLK_A, NWP), lambda i: (i, 0)),
        ],
        out_specs=[
            pl.BlockSpec((NBLK_A, K), lambda i: (i, 0)),
            pl.BlockSpec((NBLK_A, K), lambda i: (i, 0)),
        ],
        out_shape=[
            jax.ShapeDtypeStruct((N, K), jnp.int32),
            jax.ShapeDtypeStruct((N, K), jnp.float32),
        ],
    )(dy2, dx2)

    idx_flat = idx.reshape(N * K, 1)
    r_flat = r.reshape(N * K, 1)

    full = lambda shape: pl.BlockSpec(shape, lambda i: tuple(0 for _ in shape))
    phi, psi, pacc, phacc = pl.pallas_call(
        _edge_kernel,
        grid=(N * K // EBLK,),
        in_specs=[
            pl.BlockSpec((EBLK, 1), lambda i: (i, 0)),
            pl.BlockSpec((EBLK, 1), lambda i: (i, 0)),
            full((4, C)), full((1, C)), full((C, C)), full((1, C)),
            full((2, C)), full((1, C)), full((1, C)), full((1, 1)),
            full((1, NKNOTS)), full((1, NKNOTS)),
        ],
        out_specs=[
            pl.BlockSpec((EBLK, C), lambda i: (i, 0)),
            pl.BlockSpec((EBLK, 1), lambda i: (i, 0)),
            pl.BlockSpec((1, 1), lambda i: (0, 0)),
            pl.BlockSpec((1, C), lambda i: (0, 0)),
        ],
        out_shape=[
            jax.ShapeDtypeStruct((N * K, C), jnp.float32),
            jax.ShapeDtypeStruct((N * K, 1), jnp.float32),
            jax.ShapeDtypeStruct((1, 1), jnp.float32),
            jax.ShapeDtypeStruct((1, C), jnp.float32),
        ],
    )(idx_flat, r_flat, pw1, pb1.reshape(1, C), pw2, pb2.reshape(1, C),
      hw1, hb1.reshape(1, C), hw2.reshape(1, C), hb2.reshape(1, 1),
      S_m.reshape(1, NKNOTS),
      jnp.concatenate([S_m[1:], S_m[-1:]]).reshape(1, NKNOTS))

    idxcat = jnp.concatenate([idx_flat[:, 0], idx_flat[:, 0] + N])
    u = _sc_gather(x.reshape(2 * N, C), idxcat)

    nsteps = N // NBLK_C
    out = pl.pallas_call(
        _agg_kernel,
        grid=(nsteps,),
        in_specs=[
            pl.BlockSpec((EBLK, C), lambda i: (i, 0)),
            pl.BlockSpec((EBLK, C), lambda i: (i + nsteps, 0)),
            pl.BlockSpec((EBLK, C), lambda i: (i, 0)),
            pl.BlockSpec((EBLK, 1), lambda i: (i, 0)),
            pl.BlockSpec((1, 1), lambda i: (0, 0)),
            pl.BlockSpec((1, C), lambda i: (0, 0)),
            pl.BlockSpec((2, NBLK_C, C), lambda i: (0, i, 0)),
            full((C, 2 * C)), full((1, 2 * C)), full((2 * C, C)), full((1, C)),
        ],
        out_specs=pl.BlockSpec((2, NBLK_C, C), lambda i: (0, i, 0)),
        out_shape=jax.ShapeDtypeStruct((2, N, C), jnp.float32),
    )(u, u, phi, psi, pacc, phacc, x, W1, b1.reshape(1, 2 * C), W2,
      b2.reshape(1, C))

    return out
